# baseline (device time: 26047 ns/iter reference)
import jax
import jax.numpy as jnp
from jax import lax
from jax.experimental import pallas as pl
from jax.experimental.pallas import tpu as pltpu

Y_SIZE = 2
EPS = 1e-5


def kernel(x, gamma, beta):
    m, n = x.shape
    n_total = float(n * Y_SIZE)

    def body(x_ref, g_ref, b_ref, o_ref, stats_ref, recv_ref, send_sem, recv_sem):
        my_x = lax.axis_index("x")
        my_y = lax.axis_index("y")
        nbr = (my_x, 1 - my_y)

        barrier = pltpu.get_barrier_semaphore()
        pl.semaphore_signal(
            barrier, inc=1, device_id=nbr, device_id_type=pl.DeviceIdType.MESH
        )
        pl.semaphore_wait(barrier, 1)

        xv = x_ref[:, :]
        stats_ref[:, 0:1] = jnp.sum(xv, axis=1, keepdims=True)
        stats_ref[:, 1:2] = jnp.sum(xv * xv, axis=1, keepdims=True)

        rdma = pltpu.make_async_remote_copy(
            src_ref=stats_ref,
            dst_ref=recv_ref,
            send_sem=send_sem,
            recv_sem=recv_sem,
            device_id=nbr,
            device_id_type=pl.DeviceIdType.MESH,
        )
        rdma.start()
        rdma.wait()

        tot = stats_ref[:, :] + recv_ref[:, :]
        mean = tot[:, 0:1] / n_total
        var = tot[:, 1:2] / n_total - mean * mean
        inv = lax.rsqrt(var + EPS)
        o_ref[:, :] = (xv - mean) * inv * g_ref[:, :] + b_ref[:, :]

    return pl.pallas_call(
        body,
        out_shape=jax.ShapeDtypeStruct((m, n), jnp.float32),
        in_specs=[
            pl.BlockSpec(memory_space=pltpu.VMEM),
            pl.BlockSpec(memory_space=pltpu.VMEM),
            pl.BlockSpec(memory_space=pltpu.VMEM),
        ],
        out_specs=pl.BlockSpec(memory_space=pltpu.VMEM),
        scratch_shapes=[
            pltpu.VMEM((m, 2), jnp.float32),
            pltpu.VMEM((m, 2), jnp.float32),
            pltpu.SemaphoreType.DMA,
            pltpu.SemaphoreType.DMA,
        ],
        compiler_params=pltpu.CompilerParams(collective_id=0),
    )(x, gamma.reshape(1, n), beta.reshape(1, n))


# device time: 13589 ns/iter; 1.9168x vs baseline; 1.9168x over previous
import jax
import jax.numpy as jnp
from jax import lax
from jax.experimental import pallas as pl
from jax.experimental.pallas import tpu as pltpu

Y_SIZE = 2
EPS = 1e-5
NB = 4


def kernel(x, gamma, beta):
    m, n = x.shape
    bm = m // NB
    n_total = float(n * Y_SIZE)

    def moments_body(x_ref, f_ref, comm_ref, recv_ref, send_sems, recv_sems):
        my_x = lax.axis_index("x")
        my_y = lax.axis_index("y")
        nbr = (my_x, 1 - my_y)

        barrier = pltpu.get_barrier_semaphore()
        pl.semaphore_signal(
            barrier, inc=1, device_id=nbr, device_id_type=pl.DeviceIdType.MESH
        )
        pl.semaphore_wait(barrier, 1)

        rdmas = []
        for c in range(NB):
            rows = pl.ds(c * bm, bm)
            cols = pl.ds(c * bm, bm)
            xb = x_ref[rows, :]
            comm_ref[0:1, cols] = jnp.sum(xb, axis=1, keepdims=True).T
            comm_ref[1:2, cols] = jnp.sum(xb * xb, axis=1, keepdims=True).T
            rdma = pltpu.make_async_remote_copy(
                src_ref=comm_ref.at[:, cols],
                dst_ref=recv_ref.at[:, cols],
                send_sem=send_sems.at[c],
                recv_sem=recv_sems.at[c],
                device_id=nbr,
                device_id_type=pl.DeviceIdType.MESH,
            )
            rdma.start()
            rdmas.append(rdma)

        for c in range(NB):
            cols = pl.ds(c * bm, bm)
            rows = pl.ds(c * bm, bm)
            rdmas[c].wait()
            tot = comm_ref[:, cols] + recv_ref[:, cols]
            mean = tot[0:1, :] / n_total
            var = tot[1:2, :] / n_total - mean * mean
            inv = lax.rsqrt(var + EPS)
            f_ref[rows, :] = jnp.concatenate([inv, -mean * inv], axis=0).T

    factors = pl.pallas_call(
        moments_body,
        out_shape=jax.ShapeDtypeStruct((m, 2), jnp.float32),
        in_specs=[pl.BlockSpec(memory_space=pltpu.VMEM)],
        out_specs=pl.BlockSpec(memory_space=pltpu.VMEM),
        scratch_shapes=[
            pltpu.VMEM((2, m), jnp.float32),
            pltpu.VMEM((2, m), jnp.float32),
            pltpu.SemaphoreType.DMA((NB,)),
            pltpu.SemaphoreType.DMA((NB,)),
        ],
        compiler_params=pltpu.CompilerParams(collective_id=0),
    )(x)

    def normalize_body(x_ref, f_ref, g_ref, b_ref, o_hbm, o_vmem, out_sems):
        out_copies = []
        for c in range(NB):
            rows = pl.ds(c * bm, bm)
            a = f_ref[rows, 0:1]
            bb = f_ref[rows, 1:2]
            o_vmem[rows, :] = (x_ref[rows, :] * a + bb) * g_ref[:, :] + b_ref[:, :]
            cp = pltpu.make_async_copy(
                o_vmem.at[rows, :], o_hbm.at[rows, :], out_sems.at[c]
            )
            cp.start()
            out_copies.append(cp)
        for cp in out_copies:
            cp.wait()

    return pl.pallas_call(
        normalize_body,
        out_shape=jax.ShapeDtypeStruct((m, n), jnp.float32),
        in_specs=[
            pl.BlockSpec(memory_space=pltpu.VMEM),
            pl.BlockSpec(memory_space=pltpu.VMEM),
            pl.BlockSpec(memory_space=pltpu.VMEM),
            pl.BlockSpec(memory_space=pltpu.VMEM),
        ],
        out_specs=pl.BlockSpec(memory_space=pltpu.MemorySpace.HBM),
        scratch_shapes=[
            pltpu.VMEM((m, n), jnp.float32),
            pltpu.SemaphoreType.DMA((NB,)),
        ],
    )(x, factors, gamma.reshape(1, n), beta.reshape(1, n))
